# baseline (speedup 1.0000x reference)
import functools

import jax
import jax.numpy as jnp
from jax import lax
from jax.experimental import pallas as pl
from jax.experimental.pallas import tpu as pltpu
from jax.experimental.pallas import tpu_sc as plsc

N_NODES = 10000
N_PAD = 10240
N_EDGES = 320000
D = 128
N_GRAPHS = 64
N_CLASSES = 80

NC, NS = 2, 16
NW = NC * NS
EPW = N_EDGES // NW
K = 80
NCHUNK = EPW // K
RPS = N_PAD // NS
BATW = 16

RB = 10240
GRID = N_PAD // RB


def _deg_body(dst_hbm, ones_hbm, out_hbm, didx, rows, acc):
    c = lax.axis_index("c")
    s = lax.axis_index("s")
    wid = c * NS + s

    def fill(i, _):
        r = i // 8
        col = i % 8
        rows[r, pl.ds(col * 16, 16)] = jnp.zeros((16,), jnp.float32)
        return 0
    lax.fori_loop(0, K * 8, fill, 0)

    def zstripe(j, _):
        pltpu.sync_copy(rows, acc.at[pl.ds(s * RPS + j * K, K)])
        return 0
    lax.fori_loop(0, RPS // K, zstripe, 0)
    plsc.subcore_barrier()

    pltpu.sync_copy(ones_hbm, rows)
    pltpu.sync_copy(dst_hbm.at[wid], didx)

    def chunk(j, _):
        pltpu.sync_copy(rows, acc.at[didx.at[j]], add=True)
        return 0
    lax.fori_loop(0, NCHUNK, chunk, 0)
    plsc.subcore_barrier()

    pltpu.sync_copy(acc.at[pl.ds(s * RPS, RPS)],
                    out_hbm.at[c, pl.ds(s * RPS, RPS)])


@functools.lru_cache(maxsize=None)
def _deg_call_fn():
    mesh = plsc.VectorSubcoreMesh(
        core_axis_name="c", subcore_axis_name="s",
        num_cores=NC, num_subcores=NS)
    return pl.kernel(
        _deg_body,
        out_type=jax.ShapeDtypeStruct((NC, N_PAD, D), jnp.float32),
        mesh=mesh,
        scratch_types=[
            pltpu.VMEM((NCHUNK, K), jnp.int32),
            pltpu.VMEM((K, D), jnp.float32),
            pltpu.VMEM_SHARED((N_PAD, D), jnp.float32),
        ],
    )


def _deg_call(dst):
    dst3 = dst.reshape(NW, NCHUNK, K)
    return _deg_call_fn()(dst3, jnp.ones((K, D), jnp.float32))


def _scat_body(y_hbm, src3_hbm, dst3_hbm, out_hbm,
               sidx, didxa, didxb, rows0, rows1, acc, sem0, sem1, dsem):
    c = lax.axis_index("c")
    s = lax.axis_index("s")
    wid = c * NS + s

    def fill(i, _):
        r = i // 8
        col = i % 8
        rows0[r, pl.ds(col * 16, 16)] = jnp.zeros((16,), jnp.float32)
        return 0
    lax.fori_loop(0, K * 8, fill, 0)

    def zstripe(j, _):
        pltpu.sync_copy(rows0, acc.at[pl.ds(s * RPS + j * K, K)])
        return 0
    lax.fori_loop(0, RPS // K, zstripe, 0)

    pltpu.sync_copy(src3_hbm.at[wid], sidx)
    plsc.subcore_barrier()

    def g_start(j, rbuf, sem):
        pltpu.async_copy(y_hbm.at[sidx.at[j]], rbuf, sem)

    def g_wait(j, rbuf, sem):
        pltpu.make_async_copy(y_hbm.at[sidx.at[j]], rbuf, sem).wait()

    def d_start(j, dbuf):
        pltpu.async_copy(dst3_hbm.at[wid, pl.ds(j, 2)], dbuf, dsem)

    def d_wait(j, dbuf):
        pltpu.make_async_copy(dst3_hbm.at[wid, pl.ds(j, 2)], dbuf,
                              dsem).wait()

    pltpu.sync_copy(dst3_hbm.at[wid, pl.ds(0, 2)], didxa)
    d_start(2, didxb)
    g_start(0, rows0, sem0)

    def quad(q, _):
        j = q * 4
        g_start(j + 1, rows1, sem1)
        g_wait(j, rows0, sem0)
        pltpu.sync_copy(rows0, acc.at[didxa.at[0]], add=True)
        g_start(j + 2, rows0, sem0)
        g_wait(j + 1, rows1, sem1)
        pltpu.sync_copy(rows1, acc.at[didxa.at[1]], add=True)
        g_start(j + 3, rows1, sem1)
        d_wait(j + 2, didxb)
        d_start(j + 4, didxa)
        g_wait(j + 2, rows0, sem0)
        pltpu.sync_copy(rows0, acc.at[didxb.at[0]], add=True)

        @pl.when(j + 4 < NCHUNK)
        def _():
            g_start(j + 4, rows0, sem0)
        g_wait(j + 3, rows1, sem1)
        pltpu.sync_copy(rows1, acc.at[didxb.at[1]], add=True)

        @pl.when(j + 5 < NCHUNK)
        def _():
            g_start(j + 5, rows1, sem1)
        d_wait(j + 4, didxa)
        d_start(j + 6, didxb)
        return 0
    lax.fori_loop(0, NCHUNK // 4, quad, 0)

    g_wait(NCHUNK - 1, rows0, sem0)
    pltpu.sync_copy(rows0, acc.at[didxa.at[0]], add=True)
    pltpu.make_async_copy(dst3_hbm.at[wid, pl.ds(0, 2)], didxb, dsem).wait()
    plsc.subcore_barrier()

    pltpu.sync_copy(acc.at[pl.ds(s * RPS, RPS)],
                    out_hbm.at[c, pl.ds(s * RPS, RPS)])


@functools.lru_cache(maxsize=None)
def _scat_call_fn():
    mesh = plsc.VectorSubcoreMesh(
        core_axis_name="c", subcore_axis_name="s",
        num_cores=NC, num_subcores=NS)
    return pl.kernel(
        _scat_body,
        out_type=jax.ShapeDtypeStruct((NC, N_PAD, D), jnp.float32),
        mesh=mesh,
        scratch_types=[
            pltpu.VMEM((NCHUNK, K), jnp.int32),
            pltpu.VMEM((2, K), jnp.int32),
            pltpu.VMEM((2, K), jnp.int32),
            pltpu.VMEM((K, D), jnp.float32),
            pltpu.VMEM((K, D), jnp.float32),
            pltpu.VMEM_SHARED((N_PAD, D), jnp.float32),
            pltpu.SemaphoreType.DMA,
            pltpu.SemaphoreType.DMA,
            pltpu.SemaphoreType.DMA,
        ],
    )


def _scat_call(y, src, dst):
    src3 = src.reshape(NW, NCHUNK, K)
    dst3 = jnp.pad(dst.reshape(NW, NCHUNK, K), ((0, 0), (0, 3), (0, 0)))
    return _scat_call_fn()(y, src3, dst3)


def _prep_body(x_ref, deg_ref, w_ref, y_ref, dinv_ref):
    deg = 1.0 + deg_ref[0, :, 0:1] + deg_ref[1, :, 0:1]
    dinv = lax.rsqrt(deg)
    dinv_ref[...] = jnp.broadcast_to(dinv, dinv_ref.shape)
    y_ref[...] = jnp.dot(x_ref[...] * dinv, w_ref[...],
                         preferred_element_type=jnp.float32)


_prep_call = pl.pallas_call(
    _prep_body,
    grid=(GRID,),
    in_specs=[
        pl.BlockSpec((RB, D), lambda i: (i, 0)),
        pl.BlockSpec((NC, RB, D), lambda i: (0, i, 0)),
        pl.BlockSpec((D, D), lambda i: (0, 0)),
    ],
    out_specs=[
        pl.BlockSpec((RB, D), lambda i: (i, 0)),
        pl.BlockSpec((RB, BATW), lambda i: (i, 0)),
    ],
    out_shape=[
        jax.ShapeDtypeStruct((N_PAD, D), jnp.float32),
        jax.ShapeDtypeStruct((N_PAD, BATW), jnp.float32),
    ],
)


def _mid_body(z_ref, y_ref, dinv_ref, b_ref, w_ref, y2_ref):
    dinv = dinv_ref[:, 0:1]
    h = jnp.maximum(dinv * (z_ref[0] + z_ref[1] + y_ref[...]) + b_ref[...],
                    0.0)
    y2_ref[...] = jnp.dot(h * dinv, w_ref[...],
                          preferred_element_type=jnp.float32)


_mid_call = pl.pallas_call(
    _mid_body,
    grid=(GRID,),
    in_specs=[
        pl.BlockSpec((NC, RB, D), lambda i: (0, i, 0)),
        pl.BlockSpec((RB, D), lambda i: (i, 0)),
        pl.BlockSpec((RB, BATW), lambda i: (i, 0)),
        pl.BlockSpec((1, D), lambda i: (0, 0)),
        pl.BlockSpec((D, D), lambda i: (0, 0)),
    ],
    out_specs=pl.BlockSpec((RB, D), lambda i: (i, 0)),
    out_shape=jax.ShapeDtypeStruct((N_PAD, D), jnp.float32),
)


def _final_body(z_ref, y_ref, dinv_ref, b_ref, bat_ref, gx_ref,
                wf1_ref, bf1_ref, wf2_ref, bf2_ref, out_ref,
                sums_ref, cnts_ref):
    i = pl.program_id(0)
    dinv = dinv_ref[:, 0:1]
    h = jnp.maximum(dinv * (z_ref[0] + z_ref[1] + y_ref[...]) + b_ref[...],
                    0.0)
    gids = lax.broadcasted_iota(jnp.int32, (RB, N_GRAPHS), 1)
    onehot = (bat_ref[:, 0:1] == gids).astype(jnp.float32)

    dn = (((0,), (0,)), ((), ()))
    psum = lax.dot_general(onehot, h, dn,
                           preferred_element_type=jnp.float32)
    pcnt = lax.dot_general(onehot, jnp.ones((RB, 1), jnp.float32), dn,
                           preferred_element_type=jnp.float32)

    @pl.when(i == 0)
    def _():
        sums_ref[...] = jnp.zeros_like(sums_ref)
        cnts_ref[...] = jnp.zeros_like(cnts_ref)

    sums_ref[...] += psum
    cnts_ref[...] += pcnt

    @pl.when(i == GRID - 1)
    def _():
        mean = sums_ref[...] / jnp.maximum(cnts_ref[...], 1.0)
        comb = jnp.concatenate([gx_ref[...], mean], axis=1)
        hid = jnp.maximum(
            jnp.dot(comb, wf1_ref[...], preferred_element_type=jnp.float32)
            + bf1_ref[...], 0.0)
        out_ref[...] = (
            jnp.dot(hid, wf2_ref[...], preferred_element_type=jnp.float32)
            + bf2_ref[...])


_final_call = pl.pallas_call(
    _final_body,
    grid=(GRID,),
    in_specs=[
        pl.BlockSpec((NC, RB, D), lambda i: (0, i, 0)),
        pl.BlockSpec((RB, D), lambda i: (i, 0)),
        pl.BlockSpec((RB, BATW), lambda i: (i, 0)),
        pl.BlockSpec((1, D), lambda i: (0, 0)),
        pl.BlockSpec((RB, BATW), lambda i: (i, 0)),
        pl.BlockSpec((N_GRAPHS, D), lambda i: (0, 0)),
        pl.BlockSpec((2 * D, 1024), lambda i: (0, 0)),
        pl.BlockSpec((1, 1024), lambda i: (0, 0)),
        pl.BlockSpec((1024, N_CLASSES), lambda i: (0, 0)),
        pl.BlockSpec((1, N_CLASSES), lambda i: (0, 0)),
    ],
    out_specs=pl.BlockSpec((N_GRAPHS, N_CLASSES), lambda i: (0, 0)),
    out_shape=jax.ShapeDtypeStruct((N_GRAPHS, N_CLASSES), jnp.float32),
    scratch_shapes=[
        pltpu.VMEM((N_GRAPHS, D), jnp.float32),
        pltpu.VMEM((N_GRAPHS, 1), jnp.float32),
    ],
)


def kernel(global_x, x, edge_index, batch, W1, b1, W2, b2,
           Wf1, bf1, Wf2, bf2):
    src = edge_index[0].astype(jnp.int32)
    dst = edge_index[1].astype(jnp.int32)
    batch_p = jnp.concatenate([
        batch.astype(jnp.int32),
        jnp.full((N_PAD - N_NODES,), N_GRAPHS, jnp.int32)])
    batch16 = jnp.broadcast_to(batch_p[:, None], (N_PAD, BATW))
    x_p = jnp.pad(x, ((0, N_PAD - N_NODES), (0, 0)))

    deg_parts = _deg_call(dst)
    y1, dinv16 = _prep_call(x_p, deg_parts, W1)
    z1 = _scat_call(y1, src, dst)
    y2 = _mid_call(z1, y1, dinv16, b1.reshape(1, D), W2)
    z2 = _scat_call(y2, src, dst)
    out = _final_call(z2, y2, dinv16, b2.reshape(1, D), batch16,
                      global_x, Wf1, bf1.reshape(1, 1024), Wf2,
                      bf2.reshape(1, N_CLASSES))
    return out


# speedup vs baseline: 1.0093x; 1.0093x over previous
import functools

import jax
import jax.numpy as jnp
from jax import lax
from jax.experimental import pallas as pl
from jax.experimental.pallas import tpu as pltpu
from jax.experimental.pallas import tpu_sc as plsc

N_NODES = 10000
N_PAD = 10240
N_EDGES = 320000
D = 128
N_GRAPHS = 64
N_CLASSES = 80

NC, NS = 2, 16
NW = NC * NS
EPW = N_EDGES // NW
K = 80
NCHUNK = EPW // K
RPS = N_PAD // NS
BATW = 16

RB = 5120
GRID = N_PAD // RB


def _deg_body(dst_hbm, ones_hbm, out_hbm, didx, rows, acc):
    c = lax.axis_index("c")
    s = lax.axis_index("s")
    wid = c * NS + s

    def fill(i, _):
        r = i // 8
        col = i % 8
        rows[r, pl.ds(col * 16, 16)] = jnp.zeros((16,), jnp.float32)
        return 0
    lax.fori_loop(0, K * 8, fill, 0)

    def zstripe(j, _):
        pltpu.sync_copy(rows, acc.at[pl.ds(s * RPS + j * K, K)])
        return 0
    lax.fori_loop(0, RPS // K, zstripe, 0)
    plsc.subcore_barrier()

    pltpu.sync_copy(ones_hbm, rows)
    pltpu.sync_copy(dst_hbm.at[wid], didx)

    def chunk(j, _):
        pltpu.sync_copy(rows, acc.at[didx.at[j]], add=True)
        return 0
    lax.fori_loop(0, NCHUNK, chunk, 0)
    plsc.subcore_barrier()

    pltpu.sync_copy(acc.at[pl.ds(s * RPS, RPS)],
                    out_hbm.at[c, pl.ds(s * RPS, RPS)])


@functools.lru_cache(maxsize=None)
def _deg_call_fn():
    mesh = plsc.VectorSubcoreMesh(
        core_axis_name="c", subcore_axis_name="s",
        num_cores=NC, num_subcores=NS)
    return pl.kernel(
        _deg_body,
        out_type=jax.ShapeDtypeStruct((NC, N_PAD, D), jnp.float32),
        mesh=mesh,
        scratch_types=[
            pltpu.VMEM((NCHUNK, K), jnp.int32),
            pltpu.VMEM((K, D), jnp.float32),
            pltpu.VMEM_SHARED((N_PAD, D), jnp.float32),
        ],
    )


def _deg_call(dst):
    dst3 = dst.reshape(NW, NCHUNK, K)
    return _deg_call_fn()(dst3, jnp.ones((K, D), jnp.float32))


def _scat_body(y_hbm, src3_hbm, dst3_hbm, out_hbm,
               sidx, didxa, didxb, rows0, rows1, acc, sem0, sem1, dsem):
    c = lax.axis_index("c")
    s = lax.axis_index("s")
    wid = c * NS + s

    def fill(i, _):
        r = i // 8
        col = i % 8
        rows0[r, pl.ds(col * 16, 16)] = jnp.zeros((16,), jnp.float32)
        return 0
    lax.fori_loop(0, K * 8, fill, 0)

    def zstripe(j, _):
        pltpu.sync_copy(rows0, acc.at[pl.ds(s * RPS + j * K, K)])
        return 0
    lax.fori_loop(0, RPS // K, zstripe, 0)

    pltpu.sync_copy(src3_hbm.at[wid], sidx)
    plsc.subcore_barrier()

    def g_start(j, rbuf, sem):
        pltpu.async_copy(y_hbm.at[sidx.at[j]], rbuf, sem)

    def g_wait(j, rbuf, sem):
        pltpu.make_async_copy(y_hbm.at[sidx.at[j]], rbuf, sem).wait()

    def d_start(j, dbuf):
        pltpu.async_copy(dst3_hbm.at[wid, pl.ds(j, 2)], dbuf, dsem)

    def d_wait(j, dbuf):
        pltpu.make_async_copy(dst3_hbm.at[wid, pl.ds(j, 2)], dbuf,
                              dsem).wait()

    pltpu.sync_copy(dst3_hbm.at[wid, pl.ds(0, 2)], didxa)
    d_start(2, didxb)
    g_start(0, rows0, sem0)

    def quad(q, _):
        j = q * 4
        g_start(j + 1, rows1, sem1)
        g_wait(j, rows0, sem0)
        pltpu.sync_copy(rows0, acc.at[didxa.at[0]], add=True)
        g_start(j + 2, rows0, sem0)
        g_wait(j + 1, rows1, sem1)
        pltpu.sync_copy(rows1, acc.at[didxa.at[1]], add=True)
        g_start(j + 3, rows1, sem1)
        d_wait(j + 2, didxb)
        d_start(j + 4, didxa)
        g_wait(j + 2, rows0, sem0)
        pltpu.sync_copy(rows0, acc.at[didxb.at[0]], add=True)

        @pl.when(j + 4 < NCHUNK)
        def _():
            g_start(j + 4, rows0, sem0)
        g_wait(j + 3, rows1, sem1)
        pltpu.sync_copy(rows1, acc.at[didxb.at[1]], add=True)

        @pl.when(j + 5 < NCHUNK)
        def _():
            g_start(j + 5, rows1, sem1)
        d_wait(j + 4, didxa)
        d_start(j + 6, didxb)
        return 0
    lax.fori_loop(0, NCHUNK // 4, quad, 0)

    g_wait(NCHUNK - 1, rows0, sem0)
    pltpu.sync_copy(rows0, acc.at[didxa.at[0]], add=True)
    pltpu.make_async_copy(dst3_hbm.at[wid, pl.ds(0, 2)], didxb, dsem).wait()
    plsc.subcore_barrier()

    pltpu.sync_copy(acc.at[pl.ds(s * RPS, RPS)],
                    out_hbm.at[c, pl.ds(s * RPS, RPS)])


@functools.lru_cache(maxsize=None)
def _scat_call_fn():
    mesh = plsc.VectorSubcoreMesh(
        core_axis_name="c", subcore_axis_name="s",
        num_cores=NC, num_subcores=NS)
    return pl.kernel(
        _scat_body,
        out_type=jax.ShapeDtypeStruct((NC, N_PAD, D), jnp.float32),
        mesh=mesh,
        scratch_types=[
            pltpu.VMEM((NCHUNK, K), jnp.int32),
            pltpu.VMEM((2, K), jnp.int32),
            pltpu.VMEM((2, K), jnp.int32),
            pltpu.VMEM((K, D), jnp.float32),
            pltpu.VMEM((K, D), jnp.float32),
            pltpu.VMEM_SHARED((N_PAD, D), jnp.float32),
            pltpu.SemaphoreType.DMA,
            pltpu.SemaphoreType.DMA,
            pltpu.SemaphoreType.DMA,
        ],
    )


def _scat_call(y, src, dst):
    src3 = src.reshape(NW, NCHUNK, K)
    dst3 = jnp.pad(dst.reshape(NW, NCHUNK, K), ((0, 0), (0, 3), (0, 0)))
    return _scat_call_fn()(y, src3, dst3)


def _prep_body(x_ref, deg_ref, w_ref, y_ref, dinv_ref):
    deg = 1.0 + deg_ref[0, :, 0:1] + deg_ref[1, :, 0:1]
    dinv = lax.rsqrt(deg)
    dinv_ref[...] = jnp.broadcast_to(dinv, dinv_ref.shape)
    y_ref[...] = jnp.dot(x_ref[...] * dinv, w_ref[...],
                         preferred_element_type=jnp.float32)


_prep_call = pl.pallas_call(
    _prep_body,
    grid=(GRID,),
    in_specs=[
        pl.BlockSpec((RB, D), lambda i: (i, 0)),
        pl.BlockSpec((NC, RB, D), lambda i: (0, i, 0)),
        pl.BlockSpec((D, D), lambda i: (0, 0)),
    ],
    out_specs=[
        pl.BlockSpec((RB, D), lambda i: (i, 0)),
        pl.BlockSpec((RB, BATW), lambda i: (i, 0)),
    ],
    out_shape=[
        jax.ShapeDtypeStruct((N_PAD, D), jnp.float32),
        jax.ShapeDtypeStruct((N_PAD, BATW), jnp.float32),
    ],
)


def _mid_body(z_ref, y_ref, dinv_ref, b_ref, w_ref, y2_ref):
    dinv = dinv_ref[:, 0:1]
    h = jnp.maximum(dinv * (z_ref[0] + z_ref[1] + y_ref[...]) + b_ref[...],
                    0.0)
    y2_ref[...] = jnp.dot(h * dinv, w_ref[...],
                          preferred_element_type=jnp.float32)


_mid_call = pl.pallas_call(
    _mid_body,
    grid=(GRID,),
    in_specs=[
        pl.BlockSpec((NC, RB, D), lambda i: (0, i, 0)),
        pl.BlockSpec((RB, D), lambda i: (i, 0)),
        pl.BlockSpec((RB, BATW), lambda i: (i, 0)),
        pl.BlockSpec((1, D), lambda i: (0, 0)),
        pl.BlockSpec((D, D), lambda i: (0, 0)),
    ],
    out_specs=pl.BlockSpec((RB, D), lambda i: (i, 0)),
    out_shape=jax.ShapeDtypeStruct((N_PAD, D), jnp.float32),
)


def _final_body(z_ref, y_ref, dinv_ref, b_ref, bat_ref, gx_ref,
                wf1_ref, bf1_ref, wf2_ref, bf2_ref, out_ref,
                sums_ref, cnts_ref):
    i = pl.program_id(0)
    dinv = dinv_ref[:, 0:1]
    h = jnp.maximum(dinv * (z_ref[0] + z_ref[1] + y_ref[...]) + b_ref[...],
                    0.0)
    gids = lax.broadcasted_iota(jnp.int32, (RB, N_GRAPHS), 1)
    onehot = (bat_ref[:, 0:1] == gids).astype(jnp.float32)

    dn = (((0,), (0,)), ((), ()))
    psum = lax.dot_general(onehot, h, dn,
                           preferred_element_type=jnp.float32)
    pcnt = lax.dot_general(onehot, jnp.ones((RB, 1), jnp.float32), dn,
                           preferred_element_type=jnp.float32)

    @pl.when(i == 0)
    def _():
        sums_ref[...] = jnp.zeros_like(sums_ref)
        cnts_ref[...] = jnp.zeros_like(cnts_ref)

    sums_ref[...] += psum
    cnts_ref[...] += pcnt

    @pl.when(i == GRID - 1)
    def _():
        mean = sums_ref[...] / jnp.maximum(cnts_ref[...], 1.0)
        comb = jnp.concatenate([gx_ref[...], mean], axis=1)
        hid = jnp.maximum(
            jnp.dot(comb, wf1_ref[...], preferred_element_type=jnp.float32)
            + bf1_ref[...], 0.0)
        out_ref[...] = (
            jnp.dot(hid, wf2_ref[...], preferred_element_type=jnp.float32)
            + bf2_ref[...])


_final_call = pl.pallas_call(
    _final_body,
    grid=(GRID,),
    in_specs=[
        pl.BlockSpec((NC, RB, D), lambda i: (0, i, 0)),
        pl.BlockSpec((RB, D), lambda i: (i, 0)),
        pl.BlockSpec((RB, BATW), lambda i: (i, 0)),
        pl.BlockSpec((1, D), lambda i: (0, 0)),
        pl.BlockSpec((RB, BATW), lambda i: (i, 0)),
        pl.BlockSpec((N_GRAPHS, D), lambda i: (0, 0)),
        pl.BlockSpec((2 * D, 1024), lambda i: (0, 0)),
        pl.BlockSpec((1, 1024), lambda i: (0, 0)),
        pl.BlockSpec((1024, N_CLASSES), lambda i: (0, 0)),
        pl.BlockSpec((1, N_CLASSES), lambda i: (0, 0)),
    ],
    out_specs=pl.BlockSpec((N_GRAPHS, N_CLASSES), lambda i: (0, 0)),
    out_shape=jax.ShapeDtypeStruct((N_GRAPHS, N_CLASSES), jnp.float32),
    scratch_shapes=[
        pltpu.VMEM((N_GRAPHS, D), jnp.float32),
        pltpu.VMEM((N_GRAPHS, 1), jnp.float32),
    ],
)


def kernel(global_x, x, edge_index, batch, W1, b1, W2, b2,
           Wf1, bf1, Wf2, bf2):
    src = edge_index[0].astype(jnp.int32)
    dst = edge_index[1].astype(jnp.int32)
    batch_p = jnp.concatenate([
        batch.astype(jnp.int32),
        jnp.full((N_PAD - N_NODES,), N_GRAPHS, jnp.int32)])
    batch16 = jnp.broadcast_to(batch_p[:, None], (N_PAD, BATW))
    x_p = jnp.pad(x, ((0, N_PAD - N_NODES), (0, 0)))

    deg_parts = _deg_call(dst)
    y1, dinv16 = _prep_call(x_p, deg_parts, W1)
    z1 = _scat_call(y1, src, dst)
    y2 = _mid_call(z1, y1, dinv16, b1.reshape(1, D), W2)
    z2 = _scat_call(y2, src, dst)
    out = _final_call(z2, y2, dinv16, b2.reshape(1, D), batch16,
                      global_x, Wf1, bf1.reshape(1, 1024), Wf2,
                      bf2.reshape(1, N_CLASSES))
    return out


# speedup vs baseline: 1.0169x; 1.0076x over previous
import functools

import jax
import jax.numpy as jnp
from jax import lax
from jax.experimental import pallas as pl
from jax.experimental.pallas import tpu as pltpu
from jax.experimental.pallas import tpu_sc as plsc

N_NODES = 10000
N_PAD = 10240
N_EDGES = 320000
D = 128
N_GRAPHS = 64
N_CLASSES = 80

NC, NS = 2, 16
NW = NC * NS
EPW = N_EDGES // NW
K = 80
NCHUNK = EPW // K
RPS = N_PAD // NS
BATW = 16

RB = 5120
GRID = N_PAD // RB


def _deg_body(dst_hbm, out_hbm, didx, rows, acc):
    c = lax.axis_index("c")
    s = lax.axis_index("s")
    wid = c * NS + s

    def fill(i, _):
        r = i // 8
        col = i % 8
        rows[r, pl.ds(col * 16, 16)] = jnp.zeros((16,), jnp.float32)
        return 0
    lax.fori_loop(0, K * 8, fill, 0)

    def zstripe(j, _):
        pltpu.sync_copy(rows, acc.at[pl.ds(s * RPS + j * K, K)])
        return 0
    lax.fori_loop(0, RPS // K, zstripe, 0)
    plsc.subcore_barrier()

    def fill1(i, _):
        r = i // 8
        col = i % 8
        rows[r, pl.ds(col * 16, 16)] = jnp.full((16,), 1.0, jnp.float32)
        return 0
    lax.fori_loop(0, K * 8, fill1, 0)
    pltpu.sync_copy(dst_hbm.at[wid], didx)

    def chunk(j, _):
        pltpu.sync_copy(rows, acc.at[didx.at[j]], add=True)
        return 0
    lax.fori_loop(0, NCHUNK, chunk, 0)
    plsc.subcore_barrier()

    pltpu.sync_copy(acc.at[pl.ds(s * RPS, RPS)],
                    out_hbm.at[c, pl.ds(s * RPS, RPS)])


@functools.lru_cache(maxsize=None)
def _deg_call_fn():
    mesh = plsc.VectorSubcoreMesh(
        core_axis_name="c", subcore_axis_name="s",
        num_cores=NC, num_subcores=NS)
    return pl.kernel(
        _deg_body,
        out_type=jax.ShapeDtypeStruct((NC, N_PAD, D), jnp.float32),
        mesh=mesh,
        scratch_types=[
            pltpu.VMEM((NCHUNK, K), jnp.int32),
            pltpu.VMEM((K, D), jnp.float32),
            pltpu.VMEM_SHARED((N_PAD, D), jnp.float32),
        ],
    )


def _deg_call(dst):
    dst3 = dst.reshape(NW, NCHUNK, K)
    return _deg_call_fn()(dst3)


def _scat_body(y_hbm, src3_hbm, dst3_hbm, out_hbm,
               sidx, didxa, didxb, rows0, rows1, acc, sem0, sem1, dsem):
    c = lax.axis_index("c")
    s = lax.axis_index("s")
    wid = c * NS + s

    def fill(i, _):
        r = i // 8
        col = i % 8
        rows0[r, pl.ds(col * 16, 16)] = jnp.zeros((16,), jnp.float32)
        return 0
    lax.fori_loop(0, K * 8, fill, 0)

    def zstripe(j, _):
        pltpu.sync_copy(rows0, acc.at[pl.ds(s * RPS + j * K, K)])
        return 0
    lax.fori_loop(0, RPS // K, zstripe, 0)

    pltpu.sync_copy(src3_hbm.at[wid], sidx)
    plsc.subcore_barrier()

    def g_start(j, rbuf, sem):
        pltpu.async_copy(y_hbm.at[sidx.at[j]], rbuf, sem)

    def g_wait(j, rbuf, sem):
        pltpu.make_async_copy(y_hbm.at[sidx.at[j]], rbuf, sem).wait()

    def d_start(j, dbuf):
        pltpu.async_copy(dst3_hbm.at[wid, pl.ds(j, 2)], dbuf, dsem)

    def d_wait(j, dbuf):
        pltpu.make_async_copy(dst3_hbm.at[wid, pl.ds(j, 2)], dbuf,
                              dsem).wait()

    pltpu.sync_copy(dst3_hbm.at[wid, pl.ds(0, 2)], didxa)
    d_start(2, didxb)
    g_start(0, rows0, sem0)

    def quad(q, _):
        j = q * 4
        g_start(j + 1, rows1, sem1)
        g_wait(j, rows0, sem0)
        pltpu.sync_copy(rows0, acc.at[didxa.at[0]], add=True)
        g_start(j + 2, rows0, sem0)
        g_wait(j + 1, rows1, sem1)
        pltpu.sync_copy(rows1, acc.at[didxa.at[1]], add=True)
        g_start(j + 3, rows1, sem1)
        d_wait(j + 2, didxb)
        d_start(j + 4, didxa)
        g_wait(j + 2, rows0, sem0)
        pltpu.sync_copy(rows0, acc.at[didxb.at[0]], add=True)

        @pl.when(j + 4 < NCHUNK)
        def _():
            g_start(j + 4, rows0, sem0)
        g_wait(j + 3, rows1, sem1)
        pltpu.sync_copy(rows1, acc.at[didxb.at[1]], add=True)

        @pl.when(j + 5 < NCHUNK)
        def _():
            g_start(j + 5, rows1, sem1)
        d_wait(j + 4, didxa)
        d_start(j + 6, didxb)
        return 0
    lax.fori_loop(0, NCHUNK // 4, quad, 0)

    g_wait(NCHUNK - 1, rows0, sem0)
    pltpu.sync_copy(rows0, acc.at[didxa.at[0]], add=True)
    pltpu.make_async_copy(dst3_hbm.at[wid, pl.ds(0, 2)], didxb, dsem).wait()
    plsc.subcore_barrier()

    pltpu.sync_copy(acc.at[pl.ds(s * RPS, RPS)],
                    out_hbm.at[c, pl.ds(s * RPS, RPS)])


@functools.lru_cache(maxsize=None)
def _scat_call_fn():
    mesh = plsc.VectorSubcoreMesh(
        core_axis_name="c", subcore_axis_name="s",
        num_cores=NC, num_subcores=NS)
    return pl.kernel(
        _scat_body,
        out_type=jax.ShapeDtypeStruct((NC, N_PAD, D), jnp.float32),
        mesh=mesh,
        scratch_types=[
            pltpu.VMEM((NCHUNK, K), jnp.int32),
            pltpu.VMEM((2, K), jnp.int32),
            pltpu.VMEM((2, K), jnp.int32),
            pltpu.VMEM((K, D), jnp.float32),
            pltpu.VMEM((K, D), jnp.float32),
            pltpu.VMEM_SHARED((N_PAD, D), jnp.float32),
            pltpu.SemaphoreType.DMA,
            pltpu.SemaphoreType.DMA,
            pltpu.SemaphoreType.DMA,
        ],
    )


def _scat_call(y, src, dst):
    src3 = src.reshape(NW, NCHUNK, K)
    dst3 = jnp.pad(dst.reshape(NW, NCHUNK, K), ((0, 0), (0, 3), (0, 0)))
    return _scat_call_fn()(y, src3, dst3)


def _prep_body(x_ref, deg_ref, w_ref, y_ref, dinv_ref):
    deg = 1.0 + deg_ref[0, :, 0:1] + deg_ref[1, :, 0:1]
    dinv = lax.rsqrt(deg)
    dinv_ref[...] = jnp.broadcast_to(dinv, dinv_ref.shape)
    y_ref[...] = jnp.dot(x_ref[...] * dinv, w_ref[...],
                         preferred_element_type=jnp.float32)


_prep_call = pl.pallas_call(
    _prep_body,
    grid=(GRID,),
    in_specs=[
        pl.BlockSpec((RB, D), lambda i: (i, 0)),
        pl.BlockSpec((NC, RB, D), lambda i: (0, i, 0)),
        pl.BlockSpec((D, D), lambda i: (0, 0)),
    ],
    out_specs=[
        pl.BlockSpec((RB, D), lambda i: (i, 0)),
        pl.BlockSpec((RB, BATW), lambda i: (i, 0)),
    ],
    out_shape=[
        jax.ShapeDtypeStruct((N_PAD, D), jnp.float32),
        jax.ShapeDtypeStruct((N_PAD, BATW), jnp.float32),
    ],
)


def _mid_body(z_ref, y_ref, dinv_ref, b_ref, w_ref, y2_ref):
    dinv = dinv_ref[:, 0:1]
    h = jnp.maximum(dinv * (z_ref[0] + z_ref[1] + y_ref[...]) + b_ref[...],
                    0.0)
    y2_ref[...] = jnp.dot(h * dinv, w_ref[...],
                          preferred_element_type=jnp.float32)


_mid_call = pl.pallas_call(
    _mid_body,
    grid=(GRID,),
    in_specs=[
        pl.BlockSpec((NC, RB, D), lambda i: (0, i, 0)),
        pl.BlockSpec((RB, D), lambda i: (i, 0)),
        pl.BlockSpec((RB, BATW), lambda i: (i, 0)),
        pl.BlockSpec((1, D), lambda i: (0, 0)),
        pl.BlockSpec((D, D), lambda i: (0, 0)),
    ],
    out_specs=pl.BlockSpec((RB, D), lambda i: (i, 0)),
    out_shape=jax.ShapeDtypeStruct((N_PAD, D), jnp.float32),
)


def _final_body(z_ref, y_ref, dinv_ref, b_ref, bat_ref, gx_ref,
                wf1_ref, bf1_ref, wf2_ref, bf2_ref, out_ref,
                sums_ref, cnts_ref):
    i = pl.program_id(0)
    dinv = dinv_ref[:, 0:1]
    h = jnp.maximum(dinv * (z_ref[0] + z_ref[1] + y_ref[...]) + b_ref[...],
                    0.0)
    gids = lax.broadcasted_iota(jnp.int32, (RB, N_GRAPHS), 1)
    onehot = (bat_ref[:, 0:1] == gids).astype(jnp.float32)

    dn = (((0,), (0,)), ((), ()))
    psum = lax.dot_general(onehot, h, dn,
                           preferred_element_type=jnp.float32)
    pcnt = lax.dot_general(onehot, jnp.ones((RB, 1), jnp.float32), dn,
                           preferred_element_type=jnp.float32)

    @pl.when(i == 0)
    def _():
        sums_ref[...] = jnp.zeros_like(sums_ref)
        cnts_ref[...] = jnp.zeros_like(cnts_ref)

    sums_ref[...] += psum
    cnts_ref[...] += pcnt

    @pl.when(i == GRID - 1)
    def _():
        mean = sums_ref[...] / jnp.maximum(cnts_ref[...], 1.0)
        comb = jnp.concatenate([gx_ref[...], mean], axis=1)
        hid = jnp.maximum(
            jnp.dot(comb, wf1_ref[...], preferred_element_type=jnp.float32)
            + bf1_ref[...], 0.0)
        out_ref[...] = (
            jnp.dot(hid, wf2_ref[...], preferred_element_type=jnp.float32)
            + bf2_ref[...])


_final_call = pl.pallas_call(
    _final_body,
    grid=(GRID,),
    in_specs=[
        pl.BlockSpec((NC, RB, D), lambda i: (0, i, 0)),
        pl.BlockSpec((RB, D), lambda i: (i, 0)),
        pl.BlockSpec((RB, BATW), lambda i: (i, 0)),
        pl.BlockSpec((1, D), lambda i: (0, 0)),
        pl.BlockSpec((RB, BATW), lambda i: (i, 0)),
        pl.BlockSpec((N_GRAPHS, D), lambda i: (0, 0)),
        pl.BlockSpec((2 * D, 1024), lambda i: (0, 0)),
        pl.BlockSpec((1, 1024), lambda i: (0, 0)),
        pl.BlockSpec((1024, N_CLASSES), lambda i: (0, 0)),
        pl.BlockSpec((1, N_CLASSES), lambda i: (0, 0)),
    ],
    out_specs=pl.BlockSpec((N_GRAPHS, N_CLASSES), lambda i: (0, 0)),
    out_shape=jax.ShapeDtypeStruct((N_GRAPHS, N_CLASSES), jnp.float32),
    scratch_shapes=[
        pltpu.VMEM((N_GRAPHS, D), jnp.float32),
        pltpu.VMEM((N_GRAPHS, 1), jnp.float32),
    ],
)


def kernel(global_x, x, edge_index, batch, W1, b1, W2, b2,
           Wf1, bf1, Wf2, bf2):
    src = edge_index[0].astype(jnp.int32)
    dst = edge_index[1].astype(jnp.int32)
    batch_p = jnp.concatenate([
        batch.astype(jnp.int32),
        jnp.full((N_PAD - N_NODES,), N_GRAPHS, jnp.int32)])
    batch16 = jnp.broadcast_to(batch_p[:, None], (N_PAD, BATW))
    x_p = jnp.pad(x, ((0, N_PAD - N_NODES), (0, 0)))

    deg_parts = _deg_call(dst)
    y1, dinv16 = _prep_call(x_p, deg_parts, W1)
    z1 = _scat_call(y1, src, dst)
    y2 = _mid_call(z1, y1, dinv16, b1.reshape(1, D), W2)
    z2 = _scat_call(y2, src, dst)
    out = _final_call(z2, y2, dinv16, b2.reshape(1, D), batch16,
                      global_x, Wf1, bf1.reshape(1, 1024), Wf2,
                      bf2.reshape(1, N_CLASSES))
    return out
